# baseline (speedup 1.0000x reference)
import functools

import jax
import jax.numpy as jnp
from jax import lax
from jax.experimental import pallas as pl
from jax.experimental.pallas import tpu as pltpu
from jax.experimental.pallas import tpu_sc as plsc

_INFO = plsc.get_sparse_core_info()
_NC, _NS = _INFO.num_cores, _INFO.num_subcores
_NW = _NC * _NS


@functools.partial(jax.jit, static_argnames=("h", "b", "d"))
def _gather_t(x_t, table, *, h, b, d):
    blk = b // _NW
    tb = blk // 128
    te = d // 8
    mesh = plsc.VectorSubcoreMesh(core_axis_name="c", subcore_axis_name="s")

    @functools.partial(
        pl.kernel,
        out_type=jax.ShapeDtypeStruct((h, te, b // 128, 8, 128), jnp.float32),
        mesh=mesh,
        compiler_params=pltpu.CompilerParams(
            use_tc_tiling_on_sc=False, needs_layout_passes=False
        ),
        scratch_types=[
            pltpu.VMEM((h, blk), jnp.int32),
            [pltpu.VMEM((blk, d), jnp.float32) for _ in range(2)],
            pltpu.VMEM((te, tb, 8, 128), jnp.float32),
            [pltpu.SemaphoreType.DMA for _ in range(2)],
            pltpu.SemaphoreType.DMA,
        ],
    )
    def k(table_hbm, xt_hbm, out_hbm, idx_v, rows, trans_v, gsems, osem):
        wid = lax.axis_index("s") * _NC + lax.axis_index("c")
        base_b = wid * blk
        pltpu.sync_copy(xt_hbm.at[:, pl.ds(base_b, blk)], idx_v)
        iota = lax.iota(jnp.int32, 16)

        def start_g(c, r):
            pltpu.async_copy(table_hbm.at[idx_v.at[c]], rows[r], gsems[r])

        def wait_g(c, r):
            pltpu.make_async_copy(
                table_hbm.at[idx_v.at[c]], rows[r], gsems[r]
            ).wait()

        def start_o(c):
            pltpu.async_copy(
                trans_v, out_hbm.at[c, :, pl.ds(wid * tb, tb)], osem
            )

        def wait_o(c):
            pltpu.make_async_copy(
                trans_v, out_hbm.at[c, :, pl.ds(wid * tb, tb)], osem
            ).wait()

        def transpose(src):
            for t in range(tb):
                for bi0 in range(8):
                    bvec = iota + (t * 128 + bi0 * 16)

                    @pl.loop(0, te)
                    def _eo(eo):
                        for ei in range(8):
                            e = eo * 8 + ei
                            v = plsc.load_gather(
                                src, [bvec, jnp.broadcast_to(e, (16,))]
                            )
                            trans_v[eo, t, ei, pl.ds(bi0 * 16, 16)] = v

        start_g(0, 0)

        @pl.loop(0, h, step=2)
        def _pair(c):
            start_g(c + 1, 1)
            wait_g(c, 0)

            @pl.when(c > 0)
            def _():
                wait_o(c - 1)

            transpose(rows[0])
            start_o(c)

            @pl.when(c + 2 < h)
            def _():
                start_g(c + 2, 0)

            wait_g(c + 1, 1)
            wait_o(c)
            transpose(rows[1])
            start_o(c + 1)

        wait_o(h - 1)

    return k(table, x_t)


def kernel(x, table):
    b, h = x.shape
    _, d = table.shape
    x_t = jnp.transpose(x.astype(jnp.int32))
    p5 = _gather_t(x_t, table, h=h, b=b, d=d)
    return p5.transpose(2, 4, 0, 1, 3).reshape(b, h, d)


# speedup vs baseline: 1.4225x; 1.4225x over previous
import functools

import jax
import jax.numpy as jnp
from jax import lax
from jax.experimental import pallas as pl
from jax.experimental.pallas import tpu as pltpu
from jax.experimental.pallas import tpu_sc as plsc

_INFO = plsc.get_sparse_core_info()
_NC, _NS = _INFO.num_cores, _INFO.num_subcores
_NW = _NC * _NS


@functools.partial(jax.jit, static_argnames=("h", "b", "d"))
def _gather_t(x_t, table, *, h, b, d):
    blk = b // _NW
    tb = blk // 128
    te = d // 8
    mesh = plsc.VectorSubcoreMesh(core_axis_name="c", subcore_axis_name="s")

    @functools.partial(
        pl.kernel,
        out_type=jax.ShapeDtypeStruct((h, te, b // 128, 8, 128), jnp.float32),
        mesh=mesh,
        compiler_params=pltpu.CompilerParams(
            use_tc_tiling_on_sc=False, needs_layout_passes=False
        ),
        scratch_types=[
            pltpu.VMEM((h, blk), jnp.int32),
            [pltpu.VMEM((blk, d), jnp.float32) for _ in range(2)],
            pltpu.VMEM((te, tb, 8, 128), jnp.float32),
            [pltpu.SemaphoreType.DMA for _ in range(2)],
            pltpu.SemaphoreType.DMA,
        ],
    )
    def k(table_hbm, xt_hbm, out_hbm, idx_v, rows, trans_v, gsems, osem):
        wid = lax.axis_index("s") * _NC + lax.axis_index("c")
        base_b = wid * blk
        pltpu.sync_copy(xt_hbm.at[:, pl.ds(base_b, blk)], idx_v)
        iota = lax.iota(jnp.int32, 16)

        def start_g(c, r):
            pltpu.async_copy(table_hbm.at[idx_v.at[c]], rows[r], gsems[r])

        def wait_g(c, r):
            pltpu.make_async_copy(
                table_hbm.at[idx_v.at[c]], rows[r], gsems[r]
            ).wait()

        def start_o(c):
            pltpu.async_copy(
                trans_v, out_hbm.at[c, :, pl.ds(wid * tb, tb)], osem
            )

        def wait_o(c):
            pltpu.make_async_copy(
                trans_v, out_hbm.at[c, :, pl.ds(wid * tb, tb)], osem
            ).wait()

        def transpose(src):
            for t in range(tb):
                for bi0 in range(0, 128, 16):
                    bvec = iota + (t * 128 + bi0)

                    @plsc.parallel_loop(0, d, unroll=8)
                    def _e(e):
                        v = plsc.load_gather(src, [bvec, jnp.broadcast_to(e, (16,))])
                        trans_v[e >> 3, t, e & 7, pl.ds(bi0, 16)] = v

        start_g(0, 0)

        @pl.loop(0, h, step=2)
        def _pair(c):
            start_g(c + 1, 1)
            wait_g(c, 0)

            @pl.when(c > 0)
            def _():
                wait_o(c - 1)

            transpose(rows[0])
            start_o(c)

            @pl.when(c + 2 < h)
            def _():
                start_g(c + 2, 0)

            wait_g(c + 1, 1)
            wait_o(c)
            transpose(rows[1])
            start_o(c + 1)

        wait_o(h - 1)

    return k(table, x_t)


def kernel(x, table):
    b, h = x.shape
    _, d = table.shape
    x_t = jnp.transpose(x.astype(jnp.int32))
    p5 = _gather_t(x_t, table, h=h, b=b, d=d)
    return p5.transpose(2, 4, 0, 1, 3).reshape(b, h, d)
